# Initial kernel scaffold; baseline (speedup 1.0000x reference)
import jax
import jax.numpy as jnp
from jax.experimental import pallas as pl


def kernel(x, edge_index, edge_weight, W_self1, W_neigh1, b1, W_self2, W_neigh2, b2, We1, be1, We2, be2, Wb1, bb1, Wb2, bb2):
    raise NotImplementedError("write your pallas kernel here")


# speedup vs baseline: 4.1988x; 4.1988x over previous
import functools

import jax
import jax.numpy as jnp
from jax import lax
from jax.experimental import pallas as pl
from jax.experimental.pallas import tpu as pltpu
from jax.experimental.pallas import tpu_sc as plsc

_N = 10000
_NP = 10240
_E = 320000
_F = 128
_H = 128
_EH = 64

_NC = 2
_NS = 16
_NW = _NC * _NS
_EPW = _E // _NW
_EB = 400
_NBATCH = _EPW // _EB
_EBS = 400
_EPT = _E // _NS
_NBATCHS = _EPT // _EBS
_NPT = _NP // _NS

_f32 = jnp.float32


def _make_segsum(with_deg):
    mesh = plsc.VectorSubcoreMesh(core_axis_name="c", subcore_axis_name="s")
    out_type = [jax.ShapeDtypeStruct((_NC, _NP, _F // 2), _f32)]
    scratch = [
        pltpu.VMEM((_EBS,), jnp.int32),
        pltpu.VMEM((_EBS,), jnp.int32),
        pltpu.VMEM((_EBS, _F // 2), _f32),
        pltpu.VMEM_SHARED((_NP, _F // 2), _f32),
        pltpu.SemaphoreType.DMA,
    ]
    if with_deg:
        out_type.append(jax.ShapeDtypeStruct((_NP, 16), _f32))
        scratch += [
            pltpu.VMEM((_EBS, 16), _f32),
            pltpu.VMEM_SHARED((_NP, 16), _f32),
        ]

    def body(*refs):
        if with_deg:
            (m0_hbm, m1_hbm, row_hbm, col_hbm, zg_hbm, zd_hbm, onesd_hbm,
             g_out, d_out, ridx, cidx, rows, g_sh, sem, ones_v, d_sh) = refs
        else:
            (m0_hbm, m1_hbm, row_hbm, col_hbm, zg_hbm,
             g_out, ridx, cidx, rows, g_sh, sem) = refs
        c = lax.axis_index("c")
        s = lax.axis_index("s")

        sl = pl.ds(s * _NPT, _NPT)
        pltpu.sync_copy(zg_hbm.at[sl], g_sh.at[sl])
        if with_deg:
            pltpu.sync_copy(zd_hbm.at[sl], d_sh.at[sl])
            pltpu.sync_copy(onesd_hbm, ones_v)
        plsc.subcore_barrier()

        def step(i, carry):
            base = s * _EPT + i * _EBS
            pltpu.sync_copy(row_hbm.at[pl.ds(base, _EBS)], ridx)
            pltpu.sync_copy(col_hbm.at[pl.ds(base, _EBS)], cidx)

            @pl.when(c == 0)
            def _():
                pltpu.async_copy(m0_hbm.at[ridx], rows, sem).wait()

            @pl.when(c == 1)
            def _():
                pltpu.async_copy(m1_hbm.at[ridx], rows, sem).wait()

            pltpu.sync_copy(rows, g_sh.at[cidx], add=True)
            if with_deg:
                @pl.when(c == 0)
                def _():
                    pltpu.sync_copy(ones_v, d_sh.at[cidx], add=True)
            return carry

        lax.fori_loop(0, _NBATCHS, step, 0)
        plsc.subcore_barrier()

        pltpu.sync_copy(g_sh.at[sl], g_out.at[c, sl])
        if with_deg:
            @pl.when(c == 0)
            def _():
                pltpu.sync_copy(d_sh.at[sl], d_out.at[sl])

    out = tuple(out_type) if with_deg else out_type[0]
    return pl.kernel(body, out_type=out, mesh=mesh,
                     scratch_types=tuple(scratch),
                     compiler_params=pltpu.CompilerParams(
                         use_tc_tiling_on_sc=False,
                         needs_layout_passes=False))


_segsum_deg = _make_segsum(True)
_segsum = _make_segsum(False)


def _make_edge_mlp():
    mesh = plsc.VectorSubcoreMesh(core_axis_name="c", subcore_axis_name="s")
    out_type = jax.ShapeDtypeStruct((_E,), _f32)
    scratch = [
        pltpu.VMEM((_EB,), jnp.int32),
        pltpu.VMEM((_EB,), jnp.int32),
        pltpu.VMEM((_EB,), _f32),
        pltpu.VMEM((_EB, _EH), _f32),
        pltpu.VMEM((_EB, _EH), _f32),
        pltpu.VMEM((_EH,), _f32),
        pltpu.VMEM((_EH,), _f32),
        pltpu.VMEM((16,), _f32),
        pltpu.VMEM((_EB,), _f32),
        pltpu.SemaphoreType.DMA,
    ]

    def body(p_hbm, q_hbm, row_hbm, col_hbm, ew_hbm, wrow_hbm, we2_hbm,
             be2_hbm, out_hbm, ridx, cidx, ew_v, prows, qrows, wv, w2v,
             b2v, out_v, sem):
        c = lax.axis_index("c")
        s = lax.axis_index("s")
        wid = c * _NS + s
        pltpu.sync_copy(wrow_hbm, wv)
        pltpu.sync_copy(we2_hbm, w2v)
        pltpu.sync_copy(be2_hbm, b2v)
        wvecs = [wv[pl.ds(16 * j, 16)] for j in range(_EH // 16)]
        w2vecs = [w2v[pl.ds(16 * j, 16)] for j in range(_EH // 16)]
        b2s = b2v[pl.ds(0, 16)][0]

        def step(i, carry):
            base = wid * _EPW + i * _EB
            pltpu.sync_copy(row_hbm.at[pl.ds(base, _EB)], ridx)
            pltpu.sync_copy(col_hbm.at[pl.ds(base, _EB)], cidx)
            pltpu.sync_copy(ew_hbm.at[pl.ds(base, _EB)], ew_v)
            dp = pltpu.async_copy(p_hbm.at[ridx], prows, sem)
            dq = pltpu.async_copy(q_hbm.at[cidx], qrows, sem)
            dp.wait()
            dq.wait()

            def eblk(eb, carry2):
                e0 = eb * 16
                ids = lax.broadcasted_iota(jnp.int32, (16,), 0) + e0
                ewv = ew_v[pl.ds(e0, 16)]
                rv = ridx[pl.ds(e0, 16)]
                cv = cidx[pl.ds(e0, 16)]
                accs = [jnp.zeros((16,), _f32) for _ in range(4)]
                for k in range(_EH):
                    kk = jnp.full((16,), k, jnp.int32)
                    pk = plsc.load_gather(prows, [ids, kk])
                    qk = plsc.load_gather(qrows, [ids, kk])
                    hid = jnp.maximum(pk + qk + ewv * wvecs[k // 16][k % 16],
                                      0.0)
                    accs[k % 4] = accs[k % 4] + w2vecs[k // 16][k % 16] * hid
                logit = (accs[0] + accs[1]) + (accs[2] + accs[3]) + b2s
                lg = jnp.where(rv == cv, _f32(-1000000000.0), logit)
                out_v[pl.ds(e0, 16)] = lg
                return carry2

            lax.fori_loop(0, _EB // 16, eblk, 0)
            pltpu.sync_copy(out_v, out_hbm.at[pl.ds(base, _EB)])
            return carry

        lax.fori_loop(0, _NBATCH, step, 0)

    return pl.kernel(body, out_type=out_type, mesh=mesh,
                     scratch_types=tuple(scratch),
                     compiler_params=pltpu.CompilerParams(
                         use_tc_tiling_on_sc=False,
                         needs_layout_passes=False))


_edge_mlp = _make_edge_mlp()


_NBLK = 1280
_NGRID = _NP // _NBLK


def _dot(a, b):
    return jnp.dot(a, b, preferred_element_type=_f32,
                   precision=jax.lax.Precision.HIGHEST)


def _full(shape):
    return pl.BlockSpec(shape, lambda i: (0,) * len(shape))


def _blk(width):
    return pl.BlockSpec((_NBLK, width), lambda i: (i, 0))


def _tc1_body(x_ref, ws_ref, wn_ref, b_ref, u_out, m0_out, m1_out):
    xb = x_ref[...]
    u_out[...] = _dot(xb, ws_ref[...]) + b_ref[...]
    m = _dot(xb, wn_ref[...])
    m0_out[...] = m[:, :_F // 2]
    m1_out[...] = m[:, _F // 2:]


def _tc1(x, ws, wn, b):
    return pl.pallas_call(
        _tc1_body,
        grid=(_NGRID,),
        in_specs=[_blk(_F), _full((_F, _H)), _full((_F, _H)), _full((1, _H))],
        out_specs=[_blk(_H), _blk(_H // 2), _blk(_H // 2)],
        out_shape=[jax.ShapeDtypeStruct((_NP, _H), _f32),
                   jax.ShapeDtypeStruct((_NP, _H // 2), _f32),
                   jax.ShapeDtypeStruct((_NP, _H // 2), _f32)],
    )(x, ws, wn, b.reshape(1, _H))


def _ghalf(core):
    return pl.BlockSpec((1, _NBLK, _H // 2), lambda i: (core, i, 0))


def _combine(u_ref, g0_ref, g1_ref, d_ref):
    deg = jnp.maximum(d_ref[...][:, :1], 1.0)
    gcat = jnp.concatenate([g0_ref[0], g1_ref[0]], axis=-1)
    return jnp.maximum(u_ref[...] + gcat / deg, 0.0)


def _tc2_body(u_ref, g0_ref, g1_ref, d_ref, wn_ref, ws_ref, b_ref,
              u_out, m0_out, m1_out):
    h = _combine(u_ref, g0_ref, g1_ref, d_ref)
    u_out[...] = _dot(h, ws_ref[...]) + b_ref[...]
    m = _dot(h, wn_ref[...])
    m0_out[...] = m[:, :_F // 2]
    m1_out[...] = m[:, _F // 2:]


def _tc2(u1, g, d, wn2, ws2, b2):
    return pl.pallas_call(
        _tc2_body,
        grid=(_NGRID,),
        in_specs=[_blk(_H), _ghalf(0), _ghalf(1), _blk(16),
                  _full((_H, _H)), _full((_H, _H)), _full((1, _H))],
        out_specs=[_blk(_H), _blk(_H // 2), _blk(_H // 2)],
        out_shape=[jax.ShapeDtypeStruct((_NP, _H), _f32),
                   jax.ShapeDtypeStruct((_NP, _H // 2), _f32),
                   jax.ShapeDtypeStruct((_NP, _H // 2), _f32)],
    )(u1, g, g, d, wn2, ws2, b2.reshape(1, _H))


def _tc3_body(u_ref, g0_ref, g1_ref, d_ref, wea_ref, web_ref,
              be1_ref, wb1_ref, bb1_ref, wb2_ref, bb2_ref,
              p_out, q_out, bx_out):
    h = _combine(u_ref, g0_ref, g1_ref, d_ref)
    p_out[...] = _dot(h, wea_ref[...]) + be1_ref[...]
    q_out[...] = _dot(h, web_ref[...])
    bh = jnp.maximum(_dot(h, wb1_ref[...]) + bb1_ref[...], 0.0)
    bx_out[...] = _dot(bh, wb2_ref[...]) + bb2_ref[...]


def _tc3(u2, g, d, wea, web, be1, wb1, bb1, wb2, bb2):
    return pl.pallas_call(
        _tc3_body,
        grid=(_NGRID,),
        in_specs=[_blk(_H), _ghalf(0), _ghalf(1), _blk(16),
                  _full((_H, _EH)), _full((_H, _EH)), _full((1, _EH)),
                  _full((_H, _EH)), _full((1, _EH)),
                  _full((_EH, 2)), _full((1, 2))],
        out_specs=[_blk(_EH), _blk(_EH), _blk(2)],
        out_shape=[jax.ShapeDtypeStruct((_NP, _EH), _f32),
                   jax.ShapeDtypeStruct((_NP, _EH), _f32),
                   jax.ShapeDtypeStruct((_NP, 2), _f32)],
    )(u2, g, g, d, wea, web, be1.reshape(1, _EH),
      wb1, bb1.reshape(1, _EH), wb2, bb2.reshape(1, 2))


def kernel(x, edge_index, edge_weight, W_self1, W_neigh1, b1, W_self2,
           W_neigh2, b2, We1, be1, We2, be2, Wb1, bb1, Wb2, bb2):
    row = edge_index[0]
    col = edge_index[1]
    xp = jnp.pad(x, ((0, _NP - _N), (0, 0)))
    zg = jnp.zeros((_NP, _F // 2), _f32)
    zd = jnp.zeros((_NP, 16), _f32)
    onesd = jnp.zeros((_EBS, 16), _f32).at[:, 0].set(1.0)

    u1, m10, m11 = _tc1(xp, W_self1, W_neigh1, b1)
    g1, d = _segsum_deg(m10, m11, row, col, zg, zd, onesd)
    u2, m20, m21 = _tc2(u1, g1, d, W_neigh2, W_self2, b2)
    g2 = _segsum(m20, m21, row, col, zg)
    p, q, b_extra = _tc3(u2, g2, d,
                         We1[:_H], We1[_H:2 * _H], be1, Wb1, bb1, Wb2, bb2)
    wrow = We1[2 * _H]
    we2v = We2[:, 0]
    be2p = jnp.concatenate([be2, jnp.zeros((15,), _f32)])
    logits = _edge_mlp(p, q, row, col, edge_weight, wrow, we2v, be2p)
    return (logits, b_extra[:_N])
